# Initial kernel scaffold; baseline (speedup 1.0000x reference)
import jax
import jax.numpy as jnp
from jax.experimental import pallas as pl


def kernel(energy, eta, phi, track_eta, layer, edge_index):
    raise NotImplementedError("write your pallas kernel here")


# speedup vs baseline: 413.8813x; 413.8813x over previous
import functools
import math

import jax
import jax.numpy as jnp
from jax import lax
from jax.experimental import pallas as pl
from jax.experimental.pallas import tpu as pltpu
from jax.experimental.pallas import tpu_sc as plsc

NC = 2
NS = 16
LANES = 16

PI = math.pi
TWO_PI = 2.0 * math.pi
TH2 = 0.4 * 0.4

CHUNK = 2000
STAGE = 2000


def _sc_build(n_nodes: int, n_edges: int):
  acc_len = 13 * n_nodes
  n_workers = NC * NS
  ew = n_edges // n_workers
  n_chunks = ew // CHUNK
  assert ew * n_workers == n_edges and n_chunks * CHUNK == ew
  zspan = ((acc_len + NS - 1) // NS + 7) // 8 * 8
  ztail = acc_len - (NS - 1) * zspan
  assert 0 < ztail <= zspan and ztail % 8 == 0
  n_full = zspan // STAGE
  rem = zspan - n_full * STAGE
  rem_tail = ztail - n_full * STAGE
  assert rem_tail >= 0
  n_tchunks = n_nodes // STAGE
  assert n_tchunks * STAGE == n_nodes

  mesh = plsc.VectorSubcoreMesh(core_axis_name="c", subcore_axis_name="s")

  @functools.partial(
      pl.kernel,
      mesh=mesh,
      out_type=jax.ShapeDtypeStruct((NC * acc_len,), jnp.float32),
      scratch_types=dict(
          t_ep=pltpu.VMEM_SHARED((n_nodes,), jnp.float32),
          t_eta=pltpu.VMEM_SHARED((n_nodes,), jnp.float32),
          t_phi=pltpu.VMEM_SHARED((n_nodes,), jnp.float32),
          t_teta=pltpu.VMEM_SHARED((n_nodes,), jnp.float32),
          acc=pltpu.VMEM_SHARED((acc_len,), jnp.float32),
          ids_s=pltpu.VMEM((CHUNK,), jnp.int32),
          ids_d=pltpu.VMEM((CHUNK,), jnp.int32),
          g_ep=pltpu.VMEM((CHUNK,), jnp.float32),
          g_eta=pltpu.VMEM((CHUNK,), jnp.float32),
          g_phi=pltpu.VMEM((CHUNK,), jnp.float32),
          g_phid=pltpu.VMEM((CHUNK,), jnp.float32),
          g_teta=pltpu.VMEM((CHUNK,), jnp.float32),
          ones_v=pltpu.VMEM((CHUNK,), jnp.float32),
          i_sq=pltpu.VMEM((CHUNK,), jnp.int32),
          stage_f=pltpu.VMEM((STAGE,), jnp.float32),
          sem=pltpu.SemaphoreType.DMA,
      ),
  )
  def sc_kernel(ep_h, eta_h, phi_h, teta_h, src_h, dst_h, out_h, *,
                t_ep, t_eta, t_phi, t_teta, acc, ids_s, ids_d,
                g_ep, g_eta, g_phi, g_phid, g_teta, ones_v, i_sq,
                stage_f, sem):
    c = lax.axis_index("c")
    s = lax.axis_index("s")

    def zfill_body(i, carry):
      stage_f[pl.ds(i * LANES, LANES)] = jnp.zeros((LANES,), jnp.float32)
      return carry

    lax.fori_loop(0, STAGE // LANES, zfill_body, 0)

    zoff = pl.multiple_of(s * zspan, 8)
    for k in range(n_full):
      pltpu.sync_copy(stage_f,
                      acc.at[pl.ds(pl.multiple_of(zoff + k * STAGE, 8),
                                   STAGE)])

    @pl.when(s < NS - 1)
    def _():
      if rem:
        pltpu.sync_copy(stage_f.at[pl.ds(0, rem)],
                        acc.at[pl.ds(pl.multiple_of(zoff + n_full * STAGE, 8),
                                     rem)])

    @pl.when(s == NS - 1)
    def _():
      if rem_tail:
        pltpu.sync_copy(
            stage_f.at[pl.ds(0, rem_tail)],
            acc.at[pl.ds((NS - 1) * zspan + n_full * STAGE, rem_tail)])

    f_tables = [(0, ep_h, t_ep), (1, eta_h, t_eta),
                (2, phi_h, t_phi), (3, teta_h, t_teta)]
    for tile_id, hbm_ref, sp_ref in f_tables:
      @pl.when(s == tile_id)
      def _(hbm_ref=hbm_ref, sp_ref=sp_ref):
        for k in range(n_tchunks):
          pltpu.sync_copy(hbm_ref.at[pl.ds(k * STAGE, STAGE)], stage_f)
          pltpu.sync_copy(stage_f, sp_ref.at[pl.ds(k * STAGE, STAGE)])

    def ones_body(i, carry):
      ones_v[pl.ds(i * LANES, LANES)] = jnp.ones((LANES,), jnp.float32)
      return carry

    lax.fori_loop(0, CHUNK // LANES, ones_body, 0)

    plsc.subcore_barrier()

    base = (c * NS + s) * ew

    def chunk_body(g, carry):
      off = pl.multiple_of(base + g * CHUNK, 8)
      cp_is = pltpu.async_copy(src_h.at[pl.ds(off, CHUNK)], ids_s, sem)
      cp_id = pltpu.async_copy(dst_h.at[pl.ds(off, CHUNK)], ids_d, sem)
      cp_is.wait()
      cp_id.wait()
      cps = [
          pltpu.async_copy(t_ep.at[ids_s], g_ep, sem),
          pltpu.async_copy(t_eta.at[ids_s], g_eta, sem),
          pltpu.async_copy(t_phi.at[ids_s], g_phi, sem),
          pltpu.async_copy(t_phi.at[ids_d], g_phid, sem),
          pltpu.async_copy(t_teta.at[ids_d], g_teta, sem),
      ]
      cp_cnt = pltpu.async_copy(ones_v, acc.at[ids_d], sem, add=True)
      for cp in cps:
        cp.wait()

      def vec_body(i, carry2):
        sl = pl.ds(i * LANES, LANES)
        bits = lax.bitcast_convert_type(g_ep[sl], jnp.int32)
        lay = bits & 7
        e0 = lax.bitcast_convert_type(bits - lay, jnp.float32)
        deta = g_eta[sl] - g_teta[sl]
        dphi = g_phi[sl] - g_phid[sl]
        dphi = jnp.where(dphi > PI, dphi - TWO_PI, dphi)
        dphi = jnp.where(dphi < -PI, dphi + TWO_PI, dphi)
        r2 = deta * deta + dphi * dphi
        e = jnp.where(r2 < TH2, e0, 0.0)
        isum = (lay + 1) * n_nodes + ids_d[sl]
        g_ep[sl] = e
        g_eta[sl] = e * e
        ids_s[sl] = isum
        i_sq[sl] = isum + 6 * n_nodes
        return carry2

      lax.fori_loop(0, CHUNK // LANES, vec_body, 0)

      cp_sum = pltpu.async_copy(g_ep, acc.at[ids_s], sem, add=True)
      cp_sq = pltpu.async_copy(g_eta, acc.at[i_sq], sem, add=True)
      cp_cnt.wait()
      cp_sum.wait()
      cp_sq.wait()
      return carry

    lax.fori_loop(0, n_chunks, chunk_body, 0)

    plsc.subcore_barrier()

    coff = pl.multiple_of(c * acc_len + s * zspan, 8)
    for k in range(n_full):
      pltpu.sync_copy(acc.at[pl.ds(pl.multiple_of(zoff + k * STAGE, 8),
                                   STAGE)], stage_f)
      pltpu.sync_copy(stage_f,
                      out_h.at[pl.ds(pl.multiple_of(coff + k * STAGE, 8),
                                     STAGE)])

    @pl.when(s < NS - 1)
    def _():
      if rem:
        pltpu.sync_copy(acc.at[pl.ds(pl.multiple_of(zoff + n_full * STAGE, 8),
                                     rem)], stage_f.at[pl.ds(0, rem)])
        pltpu.sync_copy(stage_f.at[pl.ds(0, rem)],
                        out_h.at[pl.ds(pl.multiple_of(coff + n_full * STAGE,
                                                      8), rem)])

    @pl.when(s == NS - 1)
    def _():
      if rem_tail:
        tsrc = (NS - 1) * zspan + n_full * STAGE
        pltpu.sync_copy(acc.at[pl.ds(tsrc, rem_tail)],
                        stage_f.at[pl.ds(0, rem_tail)])
        pltpu.sync_copy(
            stage_f.at[pl.ds(0, rem_tail)],
            out_h.at[pl.ds(pl.multiple_of(c * acc_len + tsrc, 8), rem_tail)])

  return sc_kernel


def _tc_finish_body(p_ref, o_ref):
  p = p_ref[...]
  a = p[0] + p[1]
  cnt = a[0:1]
  sums = a[1:7]
  sqs = a[7:13]
  cnt_safe = jnp.maximum(cnt, 1.0)
  var = (sqs - sums * sums / cnt_safe) / jnp.maximum(cnt - 1.0, 1.0)
  std = jnp.where(cnt > 1.0, jnp.sqrt(jnp.maximum(var, 1e-12)), 0.0)
  o_ref[...] = jnp.concatenate([sums, std], axis=0)


def _tc_finish(partials, n_nodes):
  nb = 2048
  grid = (n_nodes + nb - 1) // nb
  return pl.pallas_call(
      _tc_finish_body,
      grid=(grid,),
      in_specs=[pl.BlockSpec((2, 13, nb), lambda i: (0, 0, i))],
      out_specs=pl.BlockSpec((12, nb), lambda i: (0, i)),
      out_shape=jax.ShapeDtypeStruct((12, n_nodes), jnp.float32),
  )(partials)


def kernel(energy, eta, phi, track_eta, layer, edge_index):
  n_nodes = energy.shape[0]
  n_edges = edge_index.shape[1]
  src = edge_index[0]
  dst = edge_index[1]
  e_bits = jax.lax.bitcast_convert_type(energy, jnp.int32)
  ep = jax.lax.bitcast_convert_type((e_bits & ~7) | layer.astype(jnp.int32),
                                    jnp.float32)
  sc_kernel = _sc_build(n_nodes, n_edges)
  partials = sc_kernel(ep, eta, phi, track_eta, src, dst)
  out_t = _tc_finish(partials.reshape(NC, 13, n_nodes), n_nodes)
  return out_t.T


# speedup vs baseline: 432.5415x; 1.0451x over previous
import functools
import math

import jax
import jax.numpy as jnp
from jax import lax
from jax.experimental import pallas as pl
from jax.experimental.pallas import tpu as pltpu
from jax.experimental.pallas import tpu_sc as plsc

NC = 2
NS = 16
LANES = 16

PI = math.pi
TWO_PI = 2.0 * math.pi
TH2 = 0.4 * 0.4

CHUNK = 2000
STAGE = 2000
TCH = 1000


def _sc_build(n_nodes: int, n_edges: int):
  acc_len = 13 * n_nodes
  n_workers = NC * NS
  ew = n_edges // n_workers
  n_chunks = ew // CHUNK
  assert ew * n_workers == n_edges and n_chunks * CHUNK == ew
  assert n_chunks % 2 == 0
  zspan = ((acc_len + NS - 1) // NS + 7) // 8 * 8
  ztail = acc_len - (NS - 1) * zspan
  assert 0 < ztail <= zspan and ztail % 8 == 0
  n_full = zspan // STAGE
  rem = zspan - n_full * STAGE
  rem_tail = ztail - n_full * STAGE
  assert rem_tail >= 0 and rem % 8 == 0 and rem_tail % 8 == 0
  tspan = ((n_nodes + NS - 1) // NS + 7) // 8 * 8
  ttail = n_nodes - (NS - 1) * tspan
  assert 0 < ttail <= tspan and ttail % 8 == 0
  t_full = tspan // TCH
  t_rem = tspan - t_full * TCH
  t_rem_tail = ttail - t_full * TCH
  assert t_rem_tail >= 0 and t_rem % 8 == 0 and t_rem_tail % 8 == 0

  mesh = plsc.VectorSubcoreMesh(core_axis_name="c", subcore_axis_name="s")

  @functools.partial(
      pl.kernel,
      mesh=mesh,
      out_type=jax.ShapeDtypeStruct((NC * acc_len,), jnp.float32),
      scratch_types=dict(
          t_ep=pltpu.VMEM_SHARED((n_nodes,), jnp.float32),
          t_eta=pltpu.VMEM_SHARED((n_nodes,), jnp.float32),
          t_phi=pltpu.VMEM_SHARED((n_nodes,), jnp.float32),
          t_teta=pltpu.VMEM_SHARED((n_nodes,), jnp.float32),
          acc=pltpu.VMEM_SHARED((acc_len,), jnp.float32),
          ids_s0=pltpu.VMEM((CHUNK,), jnp.int32),
          ids_d0=pltpu.VMEM((CHUNK,), jnp.int32),
          ids_s1=pltpu.VMEM((CHUNK,), jnp.int32),
          ids_d1=pltpu.VMEM((CHUNK,), jnp.int32),
          g_ep=pltpu.VMEM((CHUNK,), jnp.float32),
          g_eta=pltpu.VMEM((CHUNK,), jnp.float32),
          g_phi=pltpu.VMEM((CHUNK,), jnp.float32),
          g_phid=pltpu.VMEM((CHUNK,), jnp.float32),
          g_teta=pltpu.VMEM((CHUNK,), jnp.float32),
          ones_v=pltpu.VMEM((CHUNK,), jnp.float32),
          stage_f=pltpu.VMEM((STAGE,), jnp.float32),
          sem=pltpu.SemaphoreType.DMA,
          sem_ids=pltpu.SemaphoreType.DMA,
      ),
  )
  def sc_kernel(ep_h, eta_h, phi_h, teta_h, src_h, dst_h, out_h, *,
                t_ep, t_eta, t_phi, t_teta, acc, ids_s0, ids_d0,
                ids_s1, ids_d1, g_ep, g_eta, g_phi, g_phid, g_teta,
                ones_v, stage_f, sem, sem_ids):
    c = lax.axis_index("c")
    s = lax.axis_index("s")

    def zfill_body(i, carry):
      stage_f[pl.ds(i * LANES, LANES)] = jnp.zeros((LANES,), jnp.float32)
      return carry

    lax.fori_loop(0, STAGE // LANES, zfill_body, 0)

    zoff = pl.multiple_of(s * zspan, 8)
    for k in range(n_full):
      pltpu.sync_copy(stage_f,
                      acc.at[pl.ds(pl.multiple_of(zoff + k * STAGE, 8),
                                   STAGE)])

    @pl.when(s < NS - 1)
    def _():
      if rem:
        pltpu.sync_copy(stage_f.at[pl.ds(0, rem)],
                        acc.at[pl.ds(pl.multiple_of(zoff + n_full * STAGE, 8),
                                     rem)])

    @pl.when(s == NS - 1)
    def _():
      if rem_tail:
        pltpu.sync_copy(
            stage_f.at[pl.ds(0, rem_tail)],
            acc.at[pl.ds((NS - 1) * zspan + n_full * STAGE, rem_tail)])

    toff = pl.multiple_of(s * tspan, 8)
    for hbm_ref, sp_ref in ((ep_h, t_ep), (eta_h, t_eta),
                            (phi_h, t_phi), (teta_h, t_teta)):
      for k in range(t_full):
        o = pl.multiple_of(toff + k * TCH, 8)
        half = (k % 2) * TCH
        pltpu.sync_copy(hbm_ref.at[pl.ds(o, TCH)],
                        stage_f.at[pl.ds(half, TCH)])
        pltpu.sync_copy(stage_f.at[pl.ds(half, TCH)],
                        sp_ref.at[pl.ds(o, TCH)])

      @pl.when(s < NS - 1)
      def _(hbm_ref=hbm_ref, sp_ref=sp_ref):
        if t_rem:
          o = pl.multiple_of(toff + t_full * TCH, 8)
          pltpu.sync_copy(hbm_ref.at[pl.ds(o, t_rem)],
                          stage_f.at[pl.ds(0, t_rem)])
          pltpu.sync_copy(stage_f.at[pl.ds(0, t_rem)],
                          sp_ref.at[pl.ds(o, t_rem)])

      @pl.when(s == NS - 1)
      def _(hbm_ref=hbm_ref, sp_ref=sp_ref):
        if t_rem_tail:
          o = (NS - 1) * tspan + t_full * TCH
          pltpu.sync_copy(hbm_ref.at[pl.ds(o, t_rem_tail)],
                          stage_f.at[pl.ds(0, t_rem_tail)])
          pltpu.sync_copy(stage_f.at[pl.ds(0, t_rem_tail)],
                          sp_ref.at[pl.ds(o, t_rem_tail)])

    def ones_body(i, carry):
      ones_v[pl.ds(i * LANES, LANES)] = jnp.ones((LANES,), jnp.float32)
      return carry

    lax.fori_loop(0, CHUNK // LANES, ones_body, 0)

    plsc.subcore_barrier()

    base = (c * NS + s) * ew

    def start_ids(g, bs, bd):
      off = pl.multiple_of(base + g * CHUNK, 8)
      pltpu.async_copy(src_h.at[pl.ds(off, CHUNK)], bs, sem_ids)
      pltpu.async_copy(dst_h.at[pl.ds(off, CHUNK)], bd, sem_ids)

    def wait_ids():
      pltpu.make_async_copy(src_h.at[pl.ds(0, CHUNK)], ids_s0,
                            sem_ids).wait()
      pltpu.make_async_copy(dst_h.at[pl.ds(0, CHUNK)], ids_d0,
                            sem_ids).wait()

    def process(g, bs, bd, nbs, nbd):
      wait_ids()
      nxt = jnp.minimum(g + 1, n_chunks - 1)
      start_ids(nxt, nbs, nbd)
      cps = [
          pltpu.async_copy(t_ep.at[bs], g_ep, sem),
          pltpu.async_copy(t_eta.at[bs], g_eta, sem),
          pltpu.async_copy(t_phi.at[bs], g_phi, sem),
          pltpu.async_copy(t_phi.at[bd], g_phid, sem),
          pltpu.async_copy(t_teta.at[bd], g_teta, sem),
      ]
      for cp in cps:
        cp.wait()

      pltpu.sync_copy(ones_v, acc.at[bd], add=True)

      def vec_body(i, carry2):
        sl = pl.ds(i * LANES, LANES)
        bits = lax.bitcast_convert_type(g_ep[sl], jnp.int32)
        lay = bits & 7
        e0 = lax.bitcast_convert_type(bits - lay, jnp.float32)
        deta = g_eta[sl] - g_teta[sl]
        dphi = g_phi[sl] - g_phid[sl]
        dphi = jnp.where(dphi > PI, dphi - TWO_PI, dphi)
        dphi = jnp.where(dphi < -PI, dphi + TWO_PI, dphi)
        r2 = deta * deta + dphi * dphi
        e = jnp.where(r2 < TH2, e0, 0.0)
        isum = (lay + 1) * n_nodes + bd[sl]
        g_ep[sl] = e
        g_eta[sl] = e * e
        bs[sl] = isum
        bd[sl] = isum + 6 * n_nodes
        return carry2

      lax.fori_loop(0, CHUNK // LANES, vec_body, 0)

      pltpu.sync_copy(g_ep, acc.at[bs], add=True)
      pltpu.sync_copy(g_eta, acc.at[bd], add=True)

    start_ids(0, ids_s0, ids_d0)

    def chunk_pair(gg, carry):
      process(2 * gg, ids_s0, ids_d0, ids_s1, ids_d1)
      process(2 * gg + 1, ids_s1, ids_d1, ids_s0, ids_d0)
      return carry

    lax.fori_loop(0, n_chunks // 2, chunk_pair, 0)
    wait_ids()

    plsc.subcore_barrier()

    coff = pl.multiple_of(c * acc_len + s * zspan, 8)
    for k in range(n_full):
      pltpu.sync_copy(acc.at[pl.ds(pl.multiple_of(zoff + k * STAGE, 8),
                                   STAGE)], stage_f)
      pltpu.sync_copy(stage_f,
                      out_h.at[pl.ds(pl.multiple_of(coff + k * STAGE, 8),
                                     STAGE)])

    @pl.when(s < NS - 1)
    def _():
      if rem:
        pltpu.sync_copy(acc.at[pl.ds(pl.multiple_of(zoff + n_full * STAGE, 8),
                                     rem)], stage_f.at[pl.ds(0, rem)])
        pltpu.sync_copy(stage_f.at[pl.ds(0, rem)],
                        out_h.at[pl.ds(pl.multiple_of(coff + n_full * STAGE,
                                                      8), rem)])

    @pl.when(s == NS - 1)
    def _():
      if rem_tail:
        tsrc = (NS - 1) * zspan + n_full * STAGE
        pltpu.sync_copy(acc.at[pl.ds(tsrc, rem_tail)],
                        stage_f.at[pl.ds(0, rem_tail)])
        pltpu.sync_copy(
            stage_f.at[pl.ds(0, rem_tail)],
            out_h.at[pl.ds(pl.multiple_of(c * acc_len + tsrc, 8), rem_tail)])

  return sc_kernel


def _tc_finish_body(p_ref, o_ref):
  p = p_ref[...]
  a = p[0] + p[1]
  cnt = a[0:1]
  sums = a[1:7]
  sqs = a[7:13]
  cnt_safe = jnp.maximum(cnt, 1.0)
  var = (sqs - sums * sums / cnt_safe) / jnp.maximum(cnt - 1.0, 1.0)
  std = jnp.where(cnt > 1.0, jnp.sqrt(jnp.maximum(var, 1e-12)), 0.0)
  o_ref[...] = jnp.concatenate([sums, std], axis=0)


def _tc_finish(partials, n_nodes):
  nb = 2048
  grid = (n_nodes + nb - 1) // nb
  return pl.pallas_call(
      _tc_finish_body,
      grid=(grid,),
      in_specs=[pl.BlockSpec((2, 13, nb), lambda i: (0, 0, i))],
      out_specs=pl.BlockSpec((12, nb), lambda i: (0, i)),
      out_shape=jax.ShapeDtypeStruct((12, n_nodes), jnp.float32),
  )(partials)


def kernel(energy, eta, phi, track_eta, layer, edge_index):
  n_nodes = energy.shape[0]
  n_edges = edge_index.shape[1]
  src = edge_index[0]
  dst = edge_index[1]
  e_bits = jax.lax.bitcast_convert_type(energy, jnp.int32)
  ep = jax.lax.bitcast_convert_type((e_bits & ~7) | layer.astype(jnp.int32),
                                    jnp.float32)
  sc_kernel = _sc_build(n_nodes, n_edges)
  partials = sc_kernel(ep, eta, phi, track_eta, src, dst)
  out_t = _tc_finish(partials.reshape(NC, 13, n_nodes), n_nodes)
  return out_t.T


# speedup vs baseline: 504.2202x; 1.1657x over previous
import functools
import math

import jax
import jax.numpy as jnp
from jax import lax
from jax.experimental import pallas as pl
from jax.experimental.pallas import tpu as pltpu
from jax.experimental.pallas import tpu_sc as plsc

NC = 2
NS = 16
LANES = 16

PI = math.pi
TWO_PI = 2.0 * math.pi
TH2 = 0.4 * 0.4

CHUNK = 2000
STAGE = 2000


def _sc_build(n_nodes: int, n_edges: int):
  acc_len = 13 * n_nodes
  n_workers = NC * NS
  ew = n_edges // n_workers
  n_chunks = ew // CHUNK
  assert ew * n_workers == n_edges and n_chunks * CHUNK == ew
  zspan = ((acc_len + NS - 1) // NS + 7) // 8 * 8
  ztail = acc_len - (NS - 1) * zspan
  assert 0 < ztail <= zspan and ztail % 8 == 0
  n_full = zspan // STAGE
  rem = zspan - n_full * STAGE
  rem_tail = ztail - n_full * STAGE
  assert rem_tail >= 0
  n_tchunks = n_nodes // STAGE
  assert n_tchunks * STAGE == n_nodes

  mesh = plsc.VectorSubcoreMesh(core_axis_name="c", subcore_axis_name="s")

  @functools.partial(
      pl.kernel,
      mesh=mesh,
      out_type=jax.ShapeDtypeStruct((NC * acc_len,), jnp.float32),
      scratch_types=dict(
          t_ep=pltpu.VMEM_SHARED((n_nodes,), jnp.float32),
          t_eta=pltpu.VMEM_SHARED((n_nodes,), jnp.float32),
          t_phi=pltpu.VMEM_SHARED((n_nodes,), jnp.float32),
          t_teta=pltpu.VMEM_SHARED((n_nodes,), jnp.float32),
          acc=pltpu.VMEM_SHARED((acc_len,), jnp.float32),
          ids_s=pltpu.VMEM((CHUNK,), jnp.int32),
          ids_d=pltpu.VMEM((CHUNK,), jnp.int32),
          g_ep=pltpu.VMEM((CHUNK,), jnp.float32),
          g_eta=pltpu.VMEM((CHUNK,), jnp.float32),
          g_phi=pltpu.VMEM((CHUNK,), jnp.float32),
          g_phid=pltpu.VMEM((CHUNK,), jnp.float32),
          g_teta=pltpu.VMEM((CHUNK,), jnp.float32),
          ones_v=pltpu.VMEM((CHUNK,), jnp.float32),
          stage_f=pltpu.VMEM((STAGE,), jnp.float32),
          sem=pltpu.SemaphoreType.DMA,
      ),
  )
  def sc_kernel(ep_h, eta_h, phi_h, teta_h, src_h, dst_h, out_h, *,
                t_ep, t_eta, t_phi, t_teta, acc, ids_s, ids_d,
                g_ep, g_eta, g_phi, g_phid, g_teta, ones_v, stage_f, sem):
    c = lax.axis_index("c")
    s = lax.axis_index("s")

    def zfill_body(i, carry):
      stage_f[pl.ds(i * LANES, LANES)] = jnp.zeros((LANES,), jnp.float32)
      return carry

    lax.fori_loop(0, STAGE // LANES, zfill_body, 0)

    zoff = pl.multiple_of(s * zspan, 8)
    for k in range(n_full):
      pltpu.sync_copy(stage_f,
                      acc.at[pl.ds(pl.multiple_of(zoff + k * STAGE, 8),
                                   STAGE)])

    @pl.when(s < NS - 1)
    def _():
      if rem:
        pltpu.sync_copy(stage_f.at[pl.ds(0, rem)],
                        acc.at[pl.ds(pl.multiple_of(zoff + n_full * STAGE, 8),
                                     rem)])

    @pl.when(s == NS - 1)
    def _():
      if rem_tail:
        pltpu.sync_copy(
            stage_f.at[pl.ds(0, rem_tail)],
            acc.at[pl.ds((NS - 1) * zspan + n_full * STAGE, rem_tail)])

    f_tables = [(0, ep_h, t_ep), (1, eta_h, t_eta),
                (2, phi_h, t_phi), (3, teta_h, t_teta)]
    for tile_id, hbm_ref, sp_ref in f_tables:
      @pl.when(s == tile_id)
      def _(hbm_ref=hbm_ref, sp_ref=sp_ref):
        for k in range(n_tchunks):
          pltpu.sync_copy(hbm_ref.at[pl.ds(k * STAGE, STAGE)], stage_f)
          pltpu.sync_copy(stage_f, sp_ref.at[pl.ds(k * STAGE, STAGE)])

    def ones_body(i, carry):
      ones_v[pl.ds(i * LANES, LANES)] = jnp.ones((LANES,), jnp.float32)
      return carry

    lax.fori_loop(0, CHUNK // LANES, ones_body, 0)

    plsc.subcore_barrier()

    base = (c * NS + s) * ew

    def chunk_body(g, carry):
      off = pl.multiple_of(base + g * CHUNK, 8)
      pltpu.sync_copy(src_h.at[pl.ds(off, CHUNK)], ids_s)
      pltpu.sync_copy(dst_h.at[pl.ds(off, CHUNK)], ids_d)
      cps = [
          pltpu.async_copy(t_ep.at[ids_s], g_ep, sem),
          pltpu.async_copy(t_eta.at[ids_s], g_eta, sem),
          pltpu.async_copy(t_phi.at[ids_s], g_phi, sem),
          pltpu.async_copy(t_phi.at[ids_d], g_phid, sem),
          pltpu.async_copy(t_teta.at[ids_d], g_teta, sem),
      ]
      for cp in cps:
        cp.wait()

      pltpu.sync_copy(ones_v, acc.at[ids_d], add=True)

      @plsc.parallel_loop(0, CHUNK // LANES, unroll=4)
      def _(i):
        sl = pl.ds(i * LANES, LANES)
        bits = lax.bitcast_convert_type(g_ep[sl], jnp.int32)
        lay = bits & 7
        e0 = lax.bitcast_convert_type(bits - lay, jnp.float32)
        deta = g_eta[sl] - g_teta[sl]
        dphi = g_phi[sl] - g_phid[sl]
        dphi = jnp.where(dphi > PI, dphi - TWO_PI, dphi)
        dphi = jnp.where(dphi < -PI, dphi + TWO_PI, dphi)
        r2 = deta * deta + dphi * dphi
        e = jnp.where(r2 < TH2, e0, 0.0)
        isum = (lay + 1) * n_nodes + ids_d[sl]
        g_ep[sl] = e
        g_eta[sl] = e * e
        ids_s[sl] = isum
        ids_d[sl] = isum + 6 * n_nodes

      pltpu.sync_copy(g_ep, acc.at[ids_s], add=True)
      pltpu.sync_copy(g_eta, acc.at[ids_d], add=True)
      return carry

    lax.fori_loop(0, n_chunks, chunk_body, 0)

    plsc.subcore_barrier()

    coff = pl.multiple_of(c * acc_len + s * zspan, 8)
    for k in range(n_full):
      pltpu.sync_copy(acc.at[pl.ds(pl.multiple_of(zoff + k * STAGE, 8),
                                   STAGE)], stage_f)
      pltpu.sync_copy(stage_f,
                      out_h.at[pl.ds(pl.multiple_of(coff + k * STAGE, 8),
                                     STAGE)])

    @pl.when(s < NS - 1)
    def _():
      if rem:
        pltpu.sync_copy(acc.at[pl.ds(pl.multiple_of(zoff + n_full * STAGE, 8),
                                     rem)], stage_f.at[pl.ds(0, rem)])
        pltpu.sync_copy(stage_f.at[pl.ds(0, rem)],
                        out_h.at[pl.ds(pl.multiple_of(coff + n_full * STAGE,
                                                      8), rem)])

    @pl.when(s == NS - 1)
    def _():
      if rem_tail:
        tsrc = (NS - 1) * zspan + n_full * STAGE
        pltpu.sync_copy(acc.at[pl.ds(tsrc, rem_tail)],
                        stage_f.at[pl.ds(0, rem_tail)])
        pltpu.sync_copy(
            stage_f.at[pl.ds(0, rem_tail)],
            out_h.at[pl.ds(pl.multiple_of(c * acc_len + tsrc, 8), rem_tail)])

  return sc_kernel


def _tc_finish_body(p_ref, o_ref):
  p = p_ref[...]
  a = p[0] + p[1]
  cnt = a[0:1]
  sums = a[1:7]
  sqs = a[7:13]
  cnt_safe = jnp.maximum(cnt, 1.0)
  var = (sqs - sums * sums / cnt_safe) / jnp.maximum(cnt - 1.0, 1.0)
  std = jnp.where(cnt > 1.0, jnp.sqrt(jnp.maximum(var, 1e-12)), 0.0)
  o_ref[...] = jnp.concatenate([sums, std], axis=0)


def _tc_finish(partials, n_nodes):
  nb = 2048
  grid = (n_nodes + nb - 1) // nb
  return pl.pallas_call(
      _tc_finish_body,
      grid=(grid,),
      in_specs=[pl.BlockSpec((2, 13, nb), lambda i: (0, 0, i))],
      out_specs=pl.BlockSpec((12, nb), lambda i: (0, i)),
      out_shape=jax.ShapeDtypeStruct((12, n_nodes), jnp.float32),
  )(partials)


def kernel(energy, eta, phi, track_eta, layer, edge_index):
  n_nodes = energy.shape[0]
  n_edges = edge_index.shape[1]
  src = edge_index[0]
  dst = edge_index[1]
  e_bits = jax.lax.bitcast_convert_type(energy, jnp.int32)
  ep = jax.lax.bitcast_convert_type((e_bits & ~7) | layer.astype(jnp.int32),
                                    jnp.float32)
  sc_kernel = _sc_build(n_nodes, n_edges)
  partials = sc_kernel(ep, eta, phi, track_eta, src, dst)
  out_t = _tc_finish(partials.reshape(NC, 13, n_nodes), n_nodes)
  return out_t.T
